# initial kernel scaffold (unmeasured)
import jax
import jax.numpy as jnp
from jax import lax
from jax.experimental import pallas as pl
from jax.experimental.pallas import tpu as pltpu


def kernel(
    x,
):
    def body(*refs):
        pass

    out_shape = jax.ShapeDtypeStruct(..., jnp.float32)
    return pl.pallas_call(body, out_shape=out_shape)(...)


# baseline (device time: 119708 ns/iter reference)
import jax
import jax.numpy as jnp
from jax import lax
from jax.experimental import pallas as pl
from jax.experimental.pallas import tpu as pltpu


def kernel(x):
    m, n = x.shape
    x = x.astype(jnp.bfloat16)

    def body(x_ref, out_ref, send_sem, recv_sem):
        my_x = lax.axis_index("x")
        my_y = lax.axis_index("y")
        my_z = lax.axis_index("z")
        nbr = (my_x, my_y, 1 - my_z)

        barrier_sem = pltpu.get_barrier_semaphore()
        pl.semaphore_signal(
            barrier_sem, inc=1, device_id=nbr,
            device_id_type=pl.DeviceIdType.MESH,
        )
        pl.semaphore_wait(barrier_sem, 1)

        rdma = pltpu.make_async_remote_copy(
            src_ref=x_ref,
            dst_ref=out_ref.at[pl.ds(my_z * m, m), :],
            send_sem=send_sem,
            recv_sem=recv_sem,
            device_id=nbr,
            device_id_type=pl.DeviceIdType.MESH,
        )
        rdma.start()

        out_ref[pl.ds(my_z * m, m), :] = x_ref[...]

        rdma.wait()

    return pl.pallas_call(
        body,
        out_shape=jax.ShapeDtypeStruct((2 * m, n), jnp.bfloat16),
        in_specs=[pl.BlockSpec(memory_space=pltpu.VMEM)],
        out_specs=pl.BlockSpec(memory_space=pltpu.VMEM),
        scratch_shapes=[
            pltpu.SemaphoreType.DMA,
            pltpu.SemaphoreType.DMA,
        ],
        compiler_params=pltpu.CompilerParams(collective_id=0),
    )(x)


# device time: 81643 ns/iter; 1.4662x vs baseline; 1.4662x over previous
import jax
import jax.numpy as jnp
from jax import lax
from jax.experimental import pallas as pl
from jax.experimental.pallas import tpu as pltpu

K = 8


def kernel(x):
    m, n = x.shape
    x = x.astype(jnp.bfloat16)
    h = m // 2
    ch = h // K
    chh = ch // 2

    def body(x_ref, out_ref, zsend, zrecv, xsend, xrecv, ysend, yrecv):
        my_x = lax.axis_index("x")
        my_y = lax.axis_index("y")
        my_z = lax.axis_index("z")
        znbr = (my_x, my_y, 1 - my_z)
        xnbr = (1 - my_x, my_y, my_z)
        ynbr = (my_x, 1 - my_y, my_z)
        c = (my_x + my_y) % 2
        other_base = (1 - my_z) * m

        barrier_sem = pltpu.get_barrier_semaphore()
        for nbr in (znbr, xnbr, ynbr):
            pl.semaphore_signal(
                barrier_sem, inc=1, device_id=nbr,
                device_id_type=pl.DeviceIdType.MESH,
            )
        pl.semaphore_wait(barrier_sem, 3)

        z_rdmas = []
        for k in range(K):
            off = c * h + k * ch
            zr = pltpu.make_async_remote_copy(
                src_ref=x_ref.at[pl.ds(off, ch), :],
                dst_ref=out_ref.at[pl.ds(my_z * m + off, ch), :],
                send_sem=zsend.at[k],
                recv_sem=zrecv.at[k],
                device_id=znbr,
                device_id_type=pl.DeviceIdType.MESH,
            )
            zr.start()
            z_rdmas.append(zr)

        out_ref[pl.ds(my_z * m, m), :] = x_ref[...]

        xy_rdmas = []
        for k in range(K):
            z_rdmas[k].wait_recv()
            base = other_base + c * h + k * ch
            for nbr, sub, ssem, rsem in (
                (xnbr, 0, xsend, xrecv),
                (ynbr, chh, ysend, yrecv),
            ):
                r = pltpu.make_async_remote_copy(
                    src_ref=out_ref.at[pl.ds(base + sub, chh), :],
                    dst_ref=out_ref.at[pl.ds(base + sub, chh), :],
                    send_sem=ssem.at[k],
                    recv_sem=rsem.at[k],
                    device_id=nbr,
                    device_id_type=pl.DeviceIdType.MESH,
                )
                r.start()
                xy_rdmas.append(r)

        for k in range(K):
            z_rdmas[k].wait_send()
        for r in xy_rdmas:
            r.wait_send()
            r.wait_recv()

    return pl.pallas_call(
        body,
        out_shape=jax.ShapeDtypeStruct((2 * m, n), jnp.bfloat16),
        in_specs=[pl.BlockSpec(memory_space=pltpu.VMEM)],
        out_specs=pl.BlockSpec(memory_space=pltpu.VMEM),
        scratch_shapes=[
            pltpu.SemaphoreType.DMA((K,)),
            pltpu.SemaphoreType.DMA((K,)),
            pltpu.SemaphoreType.DMA((K,)),
            pltpu.SemaphoreType.DMA((K,)),
            pltpu.SemaphoreType.DMA((K,)),
            pltpu.SemaphoreType.DMA((K,)),
        ],
        compiler_params=pltpu.CompilerParams(collective_id=0),
    )(x)


# device time: 66974 ns/iter; 1.7874x vs baseline; 1.2190x over previous
import jax
import jax.numpy as jnp
from jax import lax
from jax.experimental import pallas as pl
from jax.experimental.pallas import tpu as pltpu

K = 8


def kernel(x):
    m, n = x.shape
    h = m // 2
    ch = h // K
    chh = ch // 2

    def body(x_ref, out_ref, zsend, zrecv, xsend, xrecv, ysend, yrecv):
        my_x = lax.axis_index("x")
        my_y = lax.axis_index("y")
        my_z = lax.axis_index("z")
        znbr = (my_x, my_y, 1 - my_z)
        xnbr = (1 - my_x, my_y, my_z)
        ynbr = (my_x, 1 - my_y, my_z)
        c = (my_x + my_y) % 2
        my_base = my_z * m
        other_base = (1 - my_z) * m

        barrier_sem = pltpu.get_barrier_semaphore()
        for nbr in (znbr, xnbr, ynbr):
            pl.semaphore_signal(
                barrier_sem, inc=1, device_id=nbr,
                device_id_type=pl.DeviceIdType.MESH,
            )
        pl.semaphore_wait(barrier_sem, 3)

        z_rdmas = []
        for k in range(K):
            off = c * h + k * ch
            out_ref[pl.ds(my_base + off, ch), :] = x_ref[
                pl.ds(off, ch), :
            ].astype(jnp.bfloat16)
            zr = pltpu.make_async_remote_copy(
                src_ref=out_ref.at[pl.ds(my_base + off, ch), :],
                dst_ref=out_ref.at[pl.ds(my_base + off, ch), :],
                send_sem=zsend.at[k],
                recv_sem=zrecv.at[k],
                device_id=znbr,
                device_id_type=pl.DeviceIdType.MESH,
            )
            zr.start()
            z_rdmas.append(zr)

        oh = (1 - c) * h
        out_ref[pl.ds(my_base + oh, h), :] = x_ref[pl.ds(oh, h), :].astype(
            jnp.bfloat16
        )

        xy_rdmas = []
        for k in range(K):
            z_rdmas[k].wait_recv()
            base = other_base + c * h + k * ch
            for nbr, sub, ssem, rsem in (
                (xnbr, 0, xsend, xrecv),
                (ynbr, chh, ysend, yrecv),
            ):
                r = pltpu.make_async_remote_copy(
                    src_ref=out_ref.at[pl.ds(base + sub, chh), :],
                    dst_ref=out_ref.at[pl.ds(base + sub, chh), :],
                    send_sem=ssem.at[k],
                    recv_sem=rsem.at[k],
                    device_id=nbr,
                    device_id_type=pl.DeviceIdType.MESH,
                )
                r.start()
                xy_rdmas.append(r)

        for k in range(K):
            z_rdmas[k].wait_send()
        for r in xy_rdmas:
            r.wait_send()
            r.wait_recv()

    return pl.pallas_call(
        body,
        out_shape=jax.ShapeDtypeStruct((2 * m, n), jnp.bfloat16),
        in_specs=[pl.BlockSpec(memory_space=pltpu.VMEM)],
        out_specs=pl.BlockSpec(memory_space=pltpu.VMEM),
        scratch_shapes=[
            pltpu.SemaphoreType.DMA((K,)),
            pltpu.SemaphoreType.DMA((K,)),
            pltpu.SemaphoreType.DMA((K,)),
            pltpu.SemaphoreType.DMA((K,)),
            pltpu.SemaphoreType.DMA((K,)),
            pltpu.SemaphoreType.DMA((K,)),
        ],
        compiler_params=pltpu.CompilerParams(collective_id=0),
    )(x)
